# baseline (speedup 1.0000x reference)
import functools
import math

import jax
import jax.numpy as jnp
from jax import lax
from jax.experimental import pallas as pl
from jax.experimental.pallas import tpu as pltpu
from jax.experimental.pallas import tpu_sc as plsc

_SMOOTHING = 0.1
_PAD_IDX = 0

_NC = 2
_NS = 16
_NW = _NC * _NS
_LANES = 16


def _sc_gather_body(pred_hbm, tgt_hbm, out_hbm, tv, idx2, vals, acc, sem, *,
                    N, V, bpw):
    wid = lax.axis_index("s") * _NC + lax.axis_index("c")
    base = wid * bpw
    cpj = 128 // _LANES

    pltpu.sync_copy(tgt_hbm.at[pl.ds(base, bpw)], tv)

    lane = lax.iota(jnp.int32, _LANES)
    for j in range(bpw // 128):
        for k in range(cpj):
            kk = j * cpj + k
            t16 = tv[pl.ds(kk * _LANES, _LANES)]
            flat = (base + kk * _LANES) * V + lane * V + t16
            idx2[j, pl.ds(k * _LANES, _LANES)] = flat
        pltpu.async_copy(pred_hbm.at[idx2.at[j]], vals.at[j], sem).wait()

    acc[...] = jnp.zeros((_LANES,), jnp.float32)
    for j in range(bpw // 128):
        for k in range(cpj):
            kk = j * cpj + k
            t16 = tv[pl.ds(kk * _LANES, _LANES)]
            g = vals.at[j][pl.ds(k * _LANES, _LANES)]
            acc[...] = acc[...] + jnp.where(t16 != _PAD_IDX, g, 0.0)

    pltpu.sync_copy(acc, out_hbm.at[wid])


def _sc_gather(pred2, tgt, N, V):
    bpw = N // _NW
    pred_flat = pred2.reshape(N * V)
    mesh = plsc.VectorSubcoreMesh(core_axis_name="c", subcore_axis_name="s")
    return pl.kernel(
        functools.partial(_sc_gather_body, N=N, V=V, bpw=bpw),
        out_type=jax.ShapeDtypeStruct((_NW, _LANES), jnp.float32),
        mesh=mesh,
        scratch_types=[
            pltpu.VMEM((bpw,), jnp.int32),
            pltpu.VMEM((bpw // 128, 128), jnp.int32),
            pltpu.VMEM((bpw // 128, 128), jnp.float32),
            pltpu.VMEM((_LANES,), jnp.float32),
            pltpu.SemaphoreType.DMA,
        ],
    )(pred_flat, tgt)


def _tc_body(tgt_ref, reward_ref, part_ref, pred_ref, out_ref, acc_ref, *,
             nsteps, V):
    i = pl.program_id(0)

    @pl.when(i == 0)
    def _init():
        acc_ref[0] = 0.0
        acc_ref[1] = 0.0

    t2 = tgt_ref[...]
    p = pred_ref[...]
    R = p.shape[0]
    valid2 = t2 != _PAD_IDX

    u = _SMOOTHING / (V - 2)
    rowsum2 = jnp.sum(p, axis=1, keepdims=True)
    p02 = p[:, 0:1]

    row_dp = u * (rowsum2 - p02)
    acc_ref[0] += jnp.sum(jnp.where(valid2, row_dp, 0.0))
    acc_ref[1] += jnp.sum(valid2.astype(jnp.float32))

    @pl.when(i == nsteps - 1)
    def _fin():
        C = (V - 2) * u * math.log(u) + (1.0 - _SMOOTHING) * math.log(1.0 - _SMOOTHING)
        G = jnp.sum(part_ref[...])
        total = acc_ref[1] * C - acc_ref[0] - (1.0 - _SMOOTHING - u) * G
        out_ref[0] = total / (nsteps * R * V) * reward_ref[0]


def kernel(pred, target, reward):
    B, S, V = pred.shape
    N = B * S
    pred2 = pred.reshape(N, V)
    tgt = target.reshape(N).astype(jnp.int32)

    partials = _sc_gather(pred2, tgt, N, V)

    R = 256
    nsteps = N // R

    out = pl.pallas_call(
        functools.partial(_tc_body, nsteps=nsteps, V=V),
        grid=(nsteps,),
        in_specs=[
            pl.BlockSpec((R, 1), lambda i: (i, 0)),
            pl.BlockSpec(memory_space=pltpu.SMEM),
            pl.BlockSpec((_NW, _LANES), lambda i: (0, 0)),
            pl.BlockSpec((R, V), lambda i: (i, 0)),
        ],
        out_specs=pl.BlockSpec(memory_space=pltpu.SMEM),
        out_shape=jax.ShapeDtypeStruct((1,), jnp.float32),
        scratch_shapes=[pltpu.SMEM((2,), jnp.float32)],
    )(tgt.reshape(N, 1), reward, partials, pred2)
    return out


# speedup vs baseline: 2.7239x; 2.7239x over previous
import functools
import math

import jax
import jax.numpy as jnp
from jax import lax
from jax.experimental import pallas as pl
from jax.experimental.pallas import tpu as pltpu

_SMOOTHING = 0.1
_PAD_IDX = 0


def _body(tgt_ref, reward_ref, pred_ref, out_ref, acc_ref, *, nsteps, V):
    i = pl.program_id(0)

    @pl.when(i == 0)
    def _init():
        acc_ref[0] = 0.0
        acc_ref[1] = 0.0

    t2 = tgt_ref[...]
    p = pred_ref[...]
    R = p.shape[0]
    valid2 = t2 != _PAD_IDX

    u = _SMOOTHING / (V - 2)
    col = lax.broadcasted_iota(jnp.int32, (R, V), 1)
    is_t = col == t2
    pt2 = jnp.sum(jnp.where(is_t, p, 0.0), axis=1, keepdims=True)
    rowsum2 = jnp.sum(p, axis=1, keepdims=True)
    p02 = p[:, 0:1]

    row_dp = u * (rowsum2 - p02 - pt2) + (1.0 - _SMOOTHING) * pt2
    dp = jnp.sum(jnp.where(valid2, row_dp, 0.0))
    nv = jnp.sum(valid2.astype(jnp.float32))

    acc_ref[0] += dp
    acc_ref[1] += nv

    @pl.when(i == nsteps - 1)
    def _fin():
        C = (V - 2) * u * math.log(u) + (1.0 - _SMOOTHING) * math.log(1.0 - _SMOOTHING)
        total = acc_ref[1] * C - acc_ref[0]
        out_ref[0] = total / (nsteps * R * V) * reward_ref[0]


def kernel(pred, target, reward):
    B, S, V = pred.shape
    N = B * S
    pred2 = pred.reshape(N, V)
    tgt = target.reshape(N, 1).astype(jnp.int32)

    R = 128
    nsteps = N // R

    out = pl.pallas_call(
        functools.partial(_body, nsteps=nsteps, V=V),
        grid=(nsteps,),
        in_specs=[
            pl.BlockSpec((R, 1), lambda i: (i, 0)),
            pl.BlockSpec(memory_space=pltpu.SMEM),
            pl.BlockSpec((R, V), lambda i: (i, 0)),
        ],
        out_specs=pl.BlockSpec(memory_space=pltpu.SMEM),
        out_shape=jax.ShapeDtypeStruct((1,), jnp.float32),
        scratch_shapes=[pltpu.SMEM((2,), jnp.float32)],
    )(tgt, reward, pred2)
    return out
